# baseline (speedup 1.0000x reference)
import functools

import jax
import jax.numpy as jnp
from jax import lax
from jax.experimental import pallas as pl
from jax.experimental.pallas import tpu as pltpu
from jax.experimental.pallas import tpu_sc as plsc

FACTOR = 32
BATCH = 16384
LANES = 16
BLOCK = 128

NC, NS = 2, 16
NW = NC * NS
B_PER_W = BATCH // NW
NGROUP = B_PER_W // LANES


def _gmf_body(user_hbm, item_hbm, tab_u, tab_i, w_hbm, b_hbm, out_hbm,
              idx_u, idx_i, slabs, vals_u, vals_i, w_v, b_v, out_v, sems):
    wid = lax.axis_index("s") * NC + lax.axis_index("c")
    base = wid * B_PER_W

    pltpu.sync_copy(user_hbm.at[wid], idx_u)
    pltpu.sync_copy(item_hbm.at[wid], idx_i)
    pltpu.sync_copy(w_hbm, w_v)
    pltpu.sync_copy(b_hbm, b_v)

    lidx = lax.iota(jnp.int32, LANES)
    bias = b_v[...]

    HALF = LANES // 2

    def gather_pass(tab, idx, vals):
        def fire(g16, par):
            vec = idx[pl.ds(g16 * LANES, LANES)]
            off = lax.shift_right_logical(vec, 7) * BLOCK
            for k in range(HALF):
                kk = par * HALF + k
                pltpu.async_copy(
                    tab.at[:, pl.ds(pl.multiple_of(off[kk], BLOCK), BLOCK)],
                    slabs.at[kk], sems.at[kk])

        def drain(par):
            for k in range(HALF):
                s = par * HALF + k
                pltpu.make_async_copy(
                    tab.at[:, pl.ds(0, BLOCK)], slabs.at[s],
                    sems.at[s]).wait()

        def extract(g16, par):
            vec = idx[pl.ds(g16 * LANES, LANES)]
            lane = vec & (BLOCK - 1)
            for k in range(HALF):
                kk = par * HALF + k
                e = g16 * LANES + kk
                cl = jnp.full((LANES,), lane[kk], jnp.int32)
                ks = jnp.full((LANES,), kk, jnp.int32)
                ce = jnp.full((LANES,), e, jnp.int32)
                lo = plsc.load_gather(slabs, [ks, lidx, cl])
                hi = plsc.load_gather(slabs, [ks, lidx + LANES, cl])
                plsc.store_scatter(vals, [lidx, ce], lo)
                plsc.store_scatter(vals, [lidx + LANES, ce], hi)

        fire(0, 0)
        fire(0, 1)

        def pair(p, carry):
            drain(0)
            extract(p, 0)

            @pl.when(p + 1 < NGROUP)
            def _():
                fire(p + 1, 0)

            drain(1)
            extract(p, 1)

            @pl.when(p + 1 < NGROUP)
            def _():
                fire(p + 1, 1)

            return carry

        lax.fori_loop(0, NGROUP, pair, 0)

    gather_pass(tab_u, idx_u, vals_u)
    gather_pass(tab_i, idx_i, vals_i)

    def compute(g, carry):
        db = pl.ds(g * LANES, LANES)
        acc = bias
        for f in range(FACTOR):
            acc = acc + vals_u[f, db] * vals_i[f, db] * w_v[f, :]
        out_v[db] = acc
        return carry

    lax.fori_loop(0, NGROUP, compute, 0)
    pltpu.sync_copy(out_v, out_hbm.at[pl.ds(base, B_PER_W)])


_gmf = functools.partial(
    pl.kernel,
    mesh=plsc.VectorSubcoreMesh(
        core_axis_name="c", subcore_axis_name="s",
        num_cores=NC, num_subcores=NS),
    out_type=jax.ShapeDtypeStruct((BATCH,), jnp.float32),
    compiler_params=pltpu.CompilerParams(
        needs_layout_passes=False, use_tc_tiling_on_sc=True,
        disable_bounds_checks=True),
    scratch_types=[
        pltpu.VMEM((B_PER_W,), jnp.int32),
        pltpu.VMEM((B_PER_W,), jnp.int32),
        pltpu.VMEM((LANES, FACTOR, BLOCK), jnp.float32),
        pltpu.VMEM((FACTOR, B_PER_W), jnp.float32),
        pltpu.VMEM((FACTOR, B_PER_W), jnp.float32),
        pltpu.VMEM((FACTOR, LANES), jnp.float32),
        pltpu.VMEM((LANES,), jnp.float32),
        pltpu.VMEM((B_PER_W,), jnp.float32),
        pltpu.SemaphoreType.DMA((LANES,)),
    ],
)(_gmf_body)


def kernel(user, item, embed_user_GMF, embed_item_GMF, predict_W, predict_b):
    user_r = user.astype(jnp.int32).reshape(NW, B_PER_W)
    item_r = item.astype(jnp.int32).reshape(NW, B_PER_W)
    w_b = jnp.broadcast_to(predict_W.reshape(FACTOR, 1), (FACTOR, LANES))
    b_b = jnp.broadcast_to(predict_b.reshape(1), (LANES,))
    return _gmf(user_r, item_r, embed_user_GMF.T, embed_item_GMF.T, w_b, b_b)


# speedup vs baseline: 1.0340x; 1.0340x over previous
import functools

import jax
import jax.numpy as jnp
from jax import lax
from jax.experimental import pallas as pl
from jax.experimental.pallas import tpu as pltpu
from jax.experimental.pallas import tpu_sc as plsc

FACTOR = 32
BATCH = 16384
LANES = 16
BLOCK = 128
HALF = 4
GENS = 4

NC, NS = 2, 16
NW = NC * NS
B_PER_W = BATCH // NW
NGROUP = B_PER_W // LANES
NCHUNK = B_PER_W // HALF
IDX_PAD = B_PER_W + LANES


def _gmf_body(user_hbm, item_hbm, tab_u, tab_i, w_hbm, b_hbm, out_hbm,
              idx_u, idx_i, slabs, vals_u, vals_i, w_v, b_v, out_v, sems):
    wid = lax.axis_index("s") * NC + lax.axis_index("c")
    base = wid * B_PER_W

    pltpu.sync_copy(user_hbm.at[wid], idx_u.at[pl.ds(0, B_PER_W)])
    pltpu.sync_copy(item_hbm.at[wid], idx_i.at[pl.ds(0, B_PER_W)])
    pltpu.sync_copy(w_hbm, w_v)
    pltpu.sync_copy(b_hbm, b_v)

    lidx = lax.iota(jnp.int32, LANES)
    bias = b_v[...]
    wv0 = w_v[pl.ds(0, LANES)]
    wv1 = w_v[pl.ds(LANES, LANES)]

    def gather_pass(tab, idx, vals):

        def fire(c, gen):
            vec = idx[pl.ds(c * HALF, LANES)]
            off = lax.shift_right_logical(vec, 7) * BLOCK
            for k in range(HALF):
                s = gen * HALF + k
                pltpu.async_copy(
                    tab.at[:, pl.ds(pl.multiple_of(off[k], BLOCK), BLOCK)],
                    slabs.at[s], sems.at[s])

        def drain(gen):
            for k in range(HALF):
                s = gen * HALF + k
                pltpu.make_async_copy(
                    tab.at[:, pl.ds(0, BLOCK)], slabs.at[s],
                    sems.at[s]).wait()

        def extract(c, gen):
            vec = idx[pl.ds(c * HALF, LANES)]
            lane = vec & (BLOCK - 1)
            for k in range(HALF):
                s = gen * HALF + k
                e = c * HALF + k
                cl = jnp.full((LANES,), lane[k], jnp.int32)
                ks = jnp.full((LANES,), s, jnp.int32)
                ce = jnp.full((LANES,), e, jnp.int32)
                lo = plsc.load_gather(slabs, [ks, lidx, cl])
                hi = plsc.load_gather(slabs, [ks, lidx + LANES, cl])
                plsc.store_scatter(vals, [lidx, ce], lo)
                plsc.store_scatter(vals, [lidx + LANES, ce], hi)

        def step(c, gen):
            drain(gen)
            extract(c, gen)

            @pl.when(c + GENS < NCHUNK)
            def _():
                fire(c + GENS, gen)

        for g in range(GENS):
            fire(g, g)

        def quad(t, carry):
            c = t * GENS
            for g in range(GENS):
                step(c + g, g)
            return carry

        lax.fori_loop(0, NCHUNK // GENS, quad, 0)

    gather_pass(tab_u, idx_u, vals_u)
    gather_pass(tab_i, idx_i, vals_i)

    def compute(g, carry):
        db = pl.ds(g * LANES, LANES)
        acc = bias
        for f in range(FACTOR):
            wf = jnp.full((LANES,), wv0[f] if f < LANES else wv1[f - LANES],
                          jnp.float32)
            acc = acc + vals_u[f, db] * vals_i[f, db] * wf
        out_v[db] = acc
        return carry

    lax.fori_loop(0, NGROUP, compute, 0)
    pltpu.sync_copy(out_v, out_hbm.at[pl.ds(base, B_PER_W)])


_gmf = functools.partial(
    pl.kernel,
    mesh=plsc.VectorSubcoreMesh(
        core_axis_name="c", subcore_axis_name="s",
        num_cores=NC, num_subcores=NS),
    out_type=jax.ShapeDtypeStruct((BATCH,), jnp.float32),
    compiler_params=pltpu.CompilerParams(
        needs_layout_passes=False, use_tc_tiling_on_sc=True,
        disable_bounds_checks=True),
    scratch_types=[
        pltpu.VMEM((IDX_PAD,), jnp.int32),
        pltpu.VMEM((IDX_PAD,), jnp.int32),
        pltpu.VMEM((GENS * HALF, FACTOR, BLOCK), jnp.float32),
        pltpu.VMEM((FACTOR, B_PER_W), jnp.float32),
        pltpu.VMEM((FACTOR, B_PER_W), jnp.float32),
        pltpu.VMEM((FACTOR,), jnp.float32),
        pltpu.VMEM((LANES,), jnp.float32),
        pltpu.VMEM((B_PER_W,), jnp.float32),
        pltpu.SemaphoreType.DMA((GENS * HALF,)),
    ],
)(_gmf_body)


def kernel(user, item, embed_user_GMF, embed_item_GMF, predict_W, predict_b):
    user_r = user.astype(jnp.int32).reshape(NW, B_PER_W)
    item_r = item.astype(jnp.int32).reshape(NW, B_PER_W)
    w_b = predict_W.reshape(FACTOR)
    b_b = jnp.broadcast_to(predict_b.reshape(1), (LANES,))
    return _gmf(user_r, item_r, embed_user_GMF.T, embed_item_GMF.T, w_b, b_b)
